# baseline (speedup 1.0000x reference)
import functools

import jax
import jax.numpy as jnp
from jax import lax
from jax.experimental import pallas as pl
from jax.experimental.pallas import tpu as pltpu
from jax.experimental.pallas import tpu_sc as plsc

BATCH = 4096
SEQ = 50
DIM = 128
HID = 1024
NCORES = 2
NSUB = 16
NWORK = NCORES * NSUB
ROWS_PER_W = BATCH // NWORK
NVREG = DIM // 16


NBUF = 8
NCHUNK = 1
CB = BATCH // NCHUNK
ROWS_PER_W_C = CB // NWORK


def _sc_pool_body(x_hbm, table_hbm, out_hbm, idx_v, rows_v, pooled_v, *sems):
    wid = lax.axis_index("s") * NCORES + lax.axis_index("c")
    base = wid * ROWS_PER_W_C
    pltpu.sync_copy(x_hbm.at[pl.ds(base, ROWS_PER_W_C)], idx_v)

    HALF = SEQ // 2

    def start(b, slot):
        pltpu.async_copy(
            table_hbm.at[idx_v.at[b, pl.ds(0, HALF)]],
            rows_v.at[slot, pl.ds(0, HALF)],
            sems[slot],
        )
        pltpu.async_copy(
            table_hbm.at[idx_v.at[b, pl.ds(HALF, SEQ - HALF)]],
            rows_v.at[slot, pl.ds(HALF, SEQ - HALF)],
            sems[slot],
        )

    def wait(slot):
        pltpu.make_async_copy(
            table_hbm.at[idx_v.at[0, pl.ds(0, HALF)]],
            rows_v.at[slot, pl.ds(0, HALF)],
            sems[slot],
        ).wait()
        pltpu.make_async_copy(
            table_hbm.at[idx_v.at[0, pl.ds(HALF, SEQ - HALF)]],
            rows_v.at[slot, pl.ds(HALF, SEQ - HALF)],
            sems[slot],
        ).wait()

    def accum(slot, b):
        def body(r, accs):
            return tuple(
                accs[k] + rows_v[slot, r, pl.ds(16 * k, 16)] for k in range(NVREG)
            )

        init = tuple(jnp.zeros((16,), jnp.float32) for _ in range(NVREG))
        accs = lax.fori_loop(0, SEQ, body, init, unroll=5)
        for k in range(NVREG):
            pooled_v[b, pl.ds(16 * k, 16)] = accs[k]

    for s in range(NBUF):
        start(s, s)

    def step(i, carry):
        b0 = i * NBUF
        for s in range(NBUF):
            wait(s)
            accum(s, b0 + s)

            @pl.when(b0 + NBUF + s < ROWS_PER_W_C)
            def _():
                start(b0 + NBUF + s, s)

        return carry

    lax.fori_loop(0, ROWS_PER_W_C // NBUF, step, 0)
    pltpu.sync_copy(pooled_v, out_hbm.at[pl.ds(base, ROWS_PER_W_C)])


def _sc_pool(x, table):
    mesh = plsc.VectorSubcoreMesh(core_axis_name="c", subcore_axis_name="s")
    return pl.kernel(
        _sc_pool_body,
        out_type=jax.ShapeDtypeStruct((CB, DIM), jnp.float32),
        mesh=mesh,
        scratch_types=[
            pltpu.VMEM((ROWS_PER_W_C, SEQ), jnp.int32),
            pltpu.VMEM((NBUF, SEQ, DIM), jnp.float32),
            pltpu.VMEM((ROWS_PER_W_C, DIM), jnp.float32),
        ]
        + [pltpu.SemaphoreType.DMA] * NBUF,
    )(x, table)


BLK = 1024


def _mlp_body(p_ref, w1_ref, b1_ref, w3_ref, b3_ref, o_ref):
    p = (p_ref[...] * (1.0 / SEQ)).astype(jnp.bfloat16)
    w1 = w1_ref[...].astype(jnp.bfloat16)
    h = jnp.dot(p, w1, preferred_element_type=jnp.float32) + b1_ref[...]
    h = jnp.maximum(h, 0.0)
    logits = jnp.dot(h, w3_ref[...], preferred_element_type=jnp.float32) + b3_ref[...]
    m = jnp.max(logits, axis=1, keepdims=True)
    lse = m + jnp.log(jnp.sum(jnp.exp(logits - m), axis=1, keepdims=True))
    o_ref[...] = logits - lse


def _mlp(pooled, w1, b1, w3, b3):
    grid = (CB // BLK,)
    return pl.pallas_call(
        _mlp_body,
        grid=grid,
        in_specs=[
            pl.BlockSpec((BLK, DIM), lambda i: (i, 0)),
            pl.BlockSpec((DIM, HID), lambda i: (0, 0)),
            pl.BlockSpec((1, HID), lambda i: (0, 0)),
            pl.BlockSpec((HID, 2), lambda i: (0, 0)),
            pl.BlockSpec((1, 2), lambda i: (0, 0)),
        ],
        out_specs=pl.BlockSpec((BLK, 2), lambda i: (i, 0)),
        out_shape=jax.ShapeDtypeStruct((CB, 2), jnp.float32),
    )(pooled, w1, b1, w3, b3)


@jax.jit
def _run(x, table, W1, b1, W3, b3):
    b1 = b1.reshape(1, HID)
    b3 = b3.reshape(1, 2)
    pooled = [_sc_pool(x[i * CB : (i + 1) * CB], table) for i in range(NCHUNK)]
    outs = [_mlp(p, W1, b1, W3, b3) for p in pooled]
    return jnp.concatenate(outs, axis=0)


def kernel(x, table, W1, b1, W3, b3):
    x = x.astype(jnp.int32)
    return _run(x, table, W1, b1, W3, b3)


# speedup vs baseline: 1.0116x; 1.0116x over previous
import functools

import jax
import jax.numpy as jnp
from jax import lax
from jax.experimental import pallas as pl
from jax.experimental.pallas import tpu as pltpu
from jax.experimental.pallas import tpu_sc as plsc

BATCH = 4096
SEQ = 50
DIM = 128
HID = 1024
NCORES = 2
NSUB = 16
NWORK = NCORES * NSUB
ROWS_PER_W = BATCH // NWORK
NVREG = DIM // 16


NBUF = 8
NCHUNK = 1
CB = BATCH // NCHUNK
ROWS_PER_W_C = CB // NWORK


def _sc_pool_body(x_hbm, table_hbm, out_hbm, idx_v, rows_v, pooled_v, *sems):
    wid = lax.axis_index("s") * NCORES + lax.axis_index("c")
    base = wid * ROWS_PER_W_C
    pltpu.sync_copy(x_hbm.at[pl.ds(base, ROWS_PER_W_C)], idx_v)

    def start(b, slot):
        pltpu.async_copy(table_hbm.at[idx_v.at[b]], rows_v.at[slot], sems[slot])

    def wait(slot):
        pltpu.make_async_copy(
            table_hbm.at[idx_v.at[0]], rows_v.at[slot], sems[slot]
        ).wait()

    def accum(slot, b):
        def body(r, accs):
            return tuple(
                accs[k] + rows_v[slot, r, pl.ds(16 * k, 16)] for k in range(NVREG)
            )

        init = tuple(jnp.zeros((16,), jnp.float32) for _ in range(NVREG))
        accs = lax.fori_loop(0, SEQ, body, init, unroll=5)
        for k in range(NVREG):
            pooled_v[b, pl.ds(16 * k, 16)] = accs[k]

    for s in range(NBUF):
        start(s, s)

    def step(i, carry):
        b0 = i * NBUF
        for s in range(NBUF):
            wait(s)
            accum(s, b0 + s)

            @pl.when(b0 + NBUF + s < ROWS_PER_W_C)
            def _():
                start(b0 + NBUF + s, s)

        return carry

    lax.fori_loop(0, ROWS_PER_W_C // NBUF, step, 0)
    pltpu.sync_copy(pooled_v, out_hbm.at[pl.ds(base, ROWS_PER_W_C)])


def _sc_pool(x, table):
    mesh = plsc.VectorSubcoreMesh(core_axis_name="c", subcore_axis_name="s")
    return pl.kernel(
        _sc_pool_body,
        out_type=jax.ShapeDtypeStruct((CB, DIM), jnp.float32),
        mesh=mesh,
        scratch_types=[
            pltpu.VMEM((ROWS_PER_W_C, SEQ), jnp.int32),
            pltpu.VMEM((NBUF, SEQ, DIM), jnp.float32),
            pltpu.VMEM((ROWS_PER_W_C, DIM), jnp.float32),
        ]
        + [pltpu.SemaphoreType.DMA] * NBUF,
    )(x, table)


BLK = 2048


def _mlp_body(p_ref, w1_ref, b1_ref, w3_ref, b3_ref, o_ref):
    p = (p_ref[...] * (1.0 / SEQ)).astype(jnp.bfloat16)
    w1 = w1_ref[...].astype(jnp.bfloat16)
    h = jnp.dot(p, w1, preferred_element_type=jnp.float32) + b1_ref[...]
    h = jnp.maximum(h, 0.0)
    logits = jnp.dot(h, w3_ref[...], preferred_element_type=jnp.float32) + b3_ref[...]
    m = jnp.max(logits, axis=1, keepdims=True)
    lse = m + jnp.log(jnp.sum(jnp.exp(logits - m), axis=1, keepdims=True))
    o_ref[...] = logits - lse


def _mlp(pooled, w1, b1, w3, b3):
    grid = (CB // BLK,)
    return pl.pallas_call(
        _mlp_body,
        grid=grid,
        in_specs=[
            pl.BlockSpec((BLK, DIM), lambda i: (i, 0)),
            pl.BlockSpec((DIM, HID), lambda i: (0, 0)),
            pl.BlockSpec((1, HID), lambda i: (0, 0)),
            pl.BlockSpec((HID, 2), lambda i: (0, 0)),
            pl.BlockSpec((1, 2), lambda i: (0, 0)),
        ],
        out_specs=pl.BlockSpec((BLK, 2), lambda i: (i, 0)),
        out_shape=jax.ShapeDtypeStruct((CB, 2), jnp.float32),
    )(pooled, w1, b1, w3, b3)


@jax.jit
def _run(x, table, W1, b1, W3, b3):
    b1 = b1.reshape(1, HID)
    b3 = b3.reshape(1, 2)
    pooled = [_sc_pool(x[i * CB : (i + 1) * CB], table) for i in range(NCHUNK)]
    outs = [_mlp(p, W1, b1, W3, b3) for p in pooled]
    return jnp.concatenate(outs, axis=0)


def kernel(x, table, W1, b1, W3, b3):
    x = x.astype(jnp.int32)
    return _run(x, table, W1, b1, W3, b3)
